# baseline (speedup 1.0000x reference)
import functools

import jax
import jax.numpy as jnp
from jax import lax
from jax.experimental import pallas as pl
from jax.experimental.pallas import tpu as pltpu
from jax.experimental.pallas import tpu_sc as plsc

N = 10000
NP = 10240
D_IN = 128
D = 64
C = 40
CP = 128
E = 320000
NC, NS = 2, 16
NW = NC * NS
CHUNK = 128
CPW = 80
E_PAD = NW * CPW * CHUNK
DUMMY = N
WDEG = 8
NBUF = 4
CPH = CPW // 2
RPT = NP // NS

_MESH = plsc.VectorSubcoreMesh(core_axis_name="c", subcore_axis_name="s")


@functools.partial(
    pl.kernel,
    out_type=jax.ShapeDtypeStruct((NP, NC * WDEG), jnp.float32),
    mesh=_MESH,
    scratch_types=[
        pltpu.VMEM((CPW, CHUNK), jnp.int32),
        pltpu.VMEM((CHUNK, WDEG), jnp.float32),
        pltpu.VMEM_SHARED((NP, WDEG), jnp.float32),
        pltpu.SemaphoreType.DMA,
    ],
    compiler_params=pltpu.CompilerParams(use_tc_tiling_on_sc=False),
)
def _deg_sc(edges_hbm, ones_hbm, zeros_hbm, out_hbm, row_v, ones_v, acc, ssem):
    c = lax.axis_index("c")
    s = lax.axis_index("s")
    w = s * NC + c
    pltpu.sync_copy(edges_hbm.at[0, pl.ds(w * CPW, CPW)], row_v)
    pltpu.sync_copy(ones_hbm, ones_v)
    pltpu.sync_copy(zeros_hbm.at[pl.ds(s * RPT, RPT)], acc.at[pl.ds(s * RPT, RPT)])
    plsc.subcore_barrier()

    def body(j, carry):
        pltpu.async_copy(ones_v, acc.at[row_v.at[j]], ssem, add=True)
        return carry

    lax.fori_loop(0, CPW, body, 0)

    def drain(j, carry):
        pltpu.make_async_copy(ones_v, acc.at[row_v.at[0]], ssem).wait()
        return carry

    lax.fori_loop(0, CPW, drain, 0)
    plsc.subcore_barrier()
    pltpu.sync_copy(acc.at[pl.ds(s * RPT, RPT)],
                    out_hbm.at[pl.ds(s * RPT, RPT), pl.ds(c * WDEG, WDEG)])


@functools.partial(
    pl.kernel,
    out_type=jax.ShapeDtypeStruct((NP, NC * D), jnp.float32),
    mesh=_MESH,
    scratch_types=[
        pltpu.VMEM((CPW, CHUNK), jnp.int32),
        pltpu.VMEM((CPW, CHUNK), jnp.int32),
        pltpu.VMEM((CHUNK, D), jnp.float32),
        pltpu.VMEM((CHUNK, D), jnp.float32),
        pltpu.VMEM_SHARED((NP, D), jnp.float32),
        pltpu.VMEM_SHARED((NP, D), jnp.float32),
        pltpu.SemaphoreType.DMA,
        pltpu.SemaphoreType.DMA,
        pltpu.SemaphoreType.DMA,
        pltpu.SemaphoreType.DMA,
    ],
    compiler_params=pltpu.CompilerParams(use_tc_tiling_on_sc=False),
)
def _agg_sc(s_hbm, edges_hbm, zeros_hbm, out_hbm,
            row_v, col_v, bufa, bufb, acc, s_sp, sema, semb, ssa, ssb):
    c = lax.axis_index("c")
    s = lax.axis_index("s")
    w = s * NC + c
    pltpu.sync_copy(edges_hbm.at[0, pl.ds(w * CPW, CPW)], row_v)
    pltpu.sync_copy(edges_hbm.at[1, pl.ds(w * CPW, CPW)], col_v)
    pltpu.sync_copy(zeros_hbm.at[pl.ds(s * RPT, RPT)], acc.at[pl.ds(s * RPT, RPT)])
    pltpu.sync_copy(s_hbm.at[pl.ds(s * RPT, RPT)], s_sp.at[pl.ds(s * RPT, RPT)])
    plsc.subcore_barrier()

    def wait_g(buf, sem):
        pltpu.make_async_copy(s_sp.at[row_v.at[0]], buf, sem).wait()

    def wait_s(buf, sem):
        pltpu.make_async_copy(buf, acc.at[col_v.at[0]], sem).wait()

    pltpu.async_copy(s_sp.at[row_v.at[0]], bufa, sema)
    pltpu.async_copy(s_sp.at[row_v.at[1]], bufb, semb)

    def body(i, carry):
        j = 2 * i
        wait_g(bufa, sema)
        pltpu.async_copy(bufa, acc.at[col_v.at[j]], ssa, add=True)
        wait_g(bufb, semb)
        pltpu.async_copy(bufb, acc.at[col_v.at[j + 1]], ssb, add=True)
        wait_s(bufa, ssa)
        pltpu.async_copy(s_sp.at[row_v.at[j + 2]], bufa, sema)
        wait_s(bufb, ssb)
        pltpu.async_copy(s_sp.at[row_v.at[j + 3]], bufb, semb)
        return carry

    lax.fori_loop(0, CPW // 2 - 1, body, 0)
    wait_g(bufa, sema)
    pltpu.async_copy(bufa, acc.at[col_v.at[CPW - 2]], ssa, add=True)
    wait_g(bufb, semb)
    pltpu.async_copy(bufb, acc.at[col_v.at[CPW - 1]], ssb, add=True)
    wait_s(bufa, ssa)
    wait_s(bufb, ssb)
    plsc.subcore_barrier()
    pltpu.sync_copy(acc.at[pl.ds(s * RPT, RPT)],
                    out_hbm.at[pl.ds(s * RPT, RPT), pl.ds(c * D, D)])


def _dis(degp_ref):
    deg = degp_ref[:, 0:1] + degp_ref[:, WDEG:WDEG + 1] + 1.0
    return lax.rsqrt(deg)


def _tc1a_body(x_ref, w1_ref, b1_ref, h_ref):
    h_ref[0:N] = lax.dot_general(
        x_ref[...], w1_ref[...], (((1,), (1,)), ((), ())),
        preferred_element_type=jnp.float32) + b1_ref[...]
    h_ref[N:] = jnp.zeros((NP - N, D), jnp.float32)


def _tc1b_body(h_ref, degp_ref, s1_ref):
    s1_ref[...] = h_ref[...] * _dis(degp_ref)


def _tc2_body(p_ref, s1_ref, degp_ref, w2_ref, b2_ref, s2_ref):
    dis = _dis(degp_ref)
    h1 = jnp.maximum(
        dis * (p_ref[:, 0:D] + p_ref[:, D:2 * D] + s1_ref[...]), 0.0)
    h = lax.dot_general(h1, w2_ref[...], (((1,), (1,)), ((), ())),
                        preferred_element_type=jnp.float32) + b2_ref[...]
    s2_ref[...] = h * dis


def _tc3_body(p_ref, s2_ref, degp_ref, w3_ref, b3_ref, out_ref):
    h2 = _dis(degp_ref) * (p_ref[:, 0:D] + p_ref[:, D:2 * D] + s2_ref[...])
    logits = lax.dot_general(h2, w3_ref[...], (((1,), (1,)), ((), ())),
                             preferred_element_type=jnp.float32) + b3_ref[...]
    m = jnp.max(logits, axis=1, keepdims=True)
    lse = m + jnp.log(jnp.sum(jnp.exp(logits - m), axis=1, keepdims=True))
    out_ref[...] = (logits - lse)[:N, :C]


_tc1a = pl.pallas_call(_tc1a_body, out_shape=jax.ShapeDtypeStruct((NP, D), jnp.float32))
_tc1b = pl.pallas_call(_tc1b_body, out_shape=jax.ShapeDtypeStruct((NP, D), jnp.float32))
_tc2 = pl.pallas_call(_tc2_body, out_shape=jax.ShapeDtypeStruct((NP, D), jnp.float32))
_tc3 = pl.pallas_call(_tc3_body, out_shape=jax.ShapeDtypeStruct((N, C), jnp.float32))


def kernel(x, edge_index, W1, b1, W2, b2, W3, b3):
    edges = jnp.pad(edge_index.astype(jnp.int32).reshape(2, E // CHUNK, CHUNK),
                    ((0, 0), (0, NW * CPW - E // CHUNK), (0, 0)),
                    constant_values=DUMMY)
    zeros_d = jnp.zeros((NP, D), jnp.float32)
    zeros_w = jnp.zeros((NP, WDEG), jnp.float32)
    ones_w = jnp.ones((CHUNK, WDEG), jnp.float32)

    h1 = _tc1a(x, W1, b1.reshape(1, D))
    degp = _deg_sc(edges, ones_w, zeros_w)
    s1 = _tc1b(h1, degp)
    p1 = _agg_sc(s1, edges, zeros_d)
    s2 = _tc2(p1, s1, degp, W2, b2.reshape(1, D))
    p2 = _agg_sc(s2, edges, zeros_d)
    w3p = jnp.pad(W3, ((0, CP - C), (0, 0)))
    b3p = jnp.concatenate(
        [b3, jnp.full((CP - C,), -1e30, jnp.float32)]).reshape(1, CP)
    return _tc3(p2, s2, degp, w3p, b3p)


# speedup vs baseline: 1.0147x; 1.0147x over previous
import functools

import jax
import jax.numpy as jnp
from jax import lax
from jax.experimental import pallas as pl
from jax.experimental.pallas import tpu as pltpu
from jax.experimental.pallas import tpu_sc as plsc

N = 10000
NP = 10240
D_IN = 128
D = 64
C = 40
CP = 128
E = 320000
NC, NS = 2, 16
NW = NC * NS
CHUNK = 128
CPW = 80
E_PAD = NW * CPW * CHUNK
DUMMY = N
WDEG = 8
KH = 24
RPT = NP // NS

_MESH = plsc.VectorSubcoreMesh(core_axis_name="c", subcore_axis_name="s")


@functools.partial(
    pl.kernel,
    out_type=jax.ShapeDtypeStruct((NP, NC * WDEG), jnp.float32),
    mesh=_MESH,
    scratch_types=[
        pltpu.VMEM((CPW, CHUNK), jnp.int32),
        pltpu.VMEM((CHUNK, WDEG), jnp.float32),
        pltpu.VMEM_SHARED((NP, WDEG), jnp.float32),
        pltpu.SemaphoreType.DMA,
    ],
    compiler_params=pltpu.CompilerParams(use_tc_tiling_on_sc=False),
)
def _deg_sc(edges_hbm, ones_hbm, zeros_hbm, out_hbm, row_v, ones_v, acc, ssem):
    c = lax.axis_index("c")
    s = lax.axis_index("s")
    w = s * NC + c
    pltpu.sync_copy(edges_hbm.at[0, pl.ds(w * CPW, CPW)], row_v)
    pltpu.sync_copy(ones_hbm, ones_v)
    pltpu.sync_copy(zeros_hbm.at[pl.ds(s * RPT, RPT)], acc.at[pl.ds(s * RPT, RPT)])
    plsc.subcore_barrier()

    def body(j, carry):
        pltpu.async_copy(ones_v, acc.at[row_v.at[j]], ssem, add=True)
        return carry

    lax.fori_loop(0, CPW, body, 0)

    def drain(j, carry):
        pltpu.make_async_copy(ones_v, acc.at[row_v.at[0]], ssem).wait()
        return carry

    lax.fori_loop(0, CPW, drain, 0)
    plsc.subcore_barrier()
    pltpu.sync_copy(acc.at[pl.ds(s * RPT, RPT)],
                    out_hbm.at[pl.ds(s * RPT, RPT), pl.ds(c * WDEG, WDEG)])


@functools.partial(
    pl.kernel,
    out_type=jax.ShapeDtypeStruct((NP, NC * D), jnp.float32),
    mesh=_MESH,
    scratch_types=[
        pltpu.VMEM((CPW, CHUNK), jnp.int32),
        pltpu.VMEM((CPW, CHUNK), jnp.int32),
        pltpu.VMEM((CHUNK, D), jnp.float32),
        pltpu.VMEM((CHUNK, D), jnp.float32),
        pltpu.VMEM_SHARED((NP, D), jnp.float32),
        pltpu.VMEM_SHARED((NP, D), jnp.float32),
        pltpu.SemaphoreType.DMA,
        pltpu.SemaphoreType.DMA,
    ],
    compiler_params=pltpu.CompilerParams(use_tc_tiling_on_sc=False),
)
def _agg_sc(s_hbm, edges_hbm, zeros_hbm, out_hbm,
            row_v, col_v, bufa, bufb, acc, s_sp, sema, semb):
    c = lax.axis_index("c")
    s = lax.axis_index("s")
    w = s * NC + c
    pltpu.sync_copy(edges_hbm.at[0, pl.ds(w * CPW, CPW)], row_v)
    pltpu.sync_copy(edges_hbm.at[1, pl.ds(w * CPW, CPW)], col_v)
    pltpu.sync_copy(zeros_hbm.at[pl.ds(s * RPT, RPT)], acc.at[pl.ds(s * RPT, RPT)])
    pltpu.sync_copy(s_hbm.at[pl.ds(s * RPT, RPT)], s_sp.at[pl.ds(s * RPT, RPT)])
    plsc.subcore_barrier()

    def wait_g(buf, sem):
        pltpu.make_async_copy(s_sp.at[row_v.at[0]], buf, sem).wait()

    def prefetch(j, buf, sem):
        @pl.when(j < KH)
        def _():
            pltpu.async_copy(s_hbm.at[row_v.at[j]], buf, sem)

        @pl.when(j >= KH)
        def _():
            pltpu.async_copy(s_sp.at[row_v.at[j]], buf, sem)

    prefetch(0, bufa, sema)
    prefetch(1, bufb, semb)

    def body(i, carry):
        j = 2 * i
        wait_g(bufa, sema)
        pltpu.sync_copy(bufa, acc.at[col_v.at[j]], add=True)
        prefetch(j + 2, bufa, sema)
        wait_g(bufb, semb)
        pltpu.sync_copy(bufb, acc.at[col_v.at[j + 1]], add=True)
        prefetch(j + 3, bufb, semb)
        return carry

    lax.fori_loop(0, CPW // 2 - 1, body, 0)
    wait_g(bufa, sema)
    pltpu.sync_copy(bufa, acc.at[col_v.at[CPW - 2]], add=True)
    wait_g(bufb, semb)
    pltpu.sync_copy(bufb, acc.at[col_v.at[CPW - 1]], add=True)
    plsc.subcore_barrier()
    pltpu.sync_copy(acc.at[pl.ds(s * RPT, RPT)],
                    out_hbm.at[pl.ds(s * RPT, RPT), pl.ds(c * D, D)])


def _dis(degp_ref):
    deg = degp_ref[:, 0:1] + degp_ref[:, WDEG:WDEG + 1] + 1.0
    return lax.rsqrt(deg)


def _tc1a_body(x_ref, w1_ref, b1_ref, h_ref):
    h_ref[0:N] = lax.dot_general(
        x_ref[...], w1_ref[...], (((1,), (1,)), ((), ())),
        preferred_element_type=jnp.float32) + b1_ref[...]
    h_ref[N:] = jnp.zeros((NP - N, D), jnp.float32)


def _tc1b_body(h_ref, degp_ref, s1_ref):
    s1_ref[...] = h_ref[...] * _dis(degp_ref)


def _tc2_body(p_ref, s1_ref, degp_ref, w2_ref, b2_ref, s2_ref):
    dis = _dis(degp_ref)
    h1 = jnp.maximum(
        dis * (p_ref[:, 0:D] + p_ref[:, D:2 * D] + s1_ref[...]), 0.0)
    h = lax.dot_general(h1, w2_ref[...], (((1,), (1,)), ((), ())),
                        preferred_element_type=jnp.float32) + b2_ref[...]
    s2_ref[...] = h * dis


def _tc3_body(p_ref, s2_ref, degp_ref, w3_ref, b3_ref, out_ref):
    h2 = _dis(degp_ref) * (p_ref[:, 0:D] + p_ref[:, D:2 * D] + s2_ref[...])
    logits = lax.dot_general(h2, w3_ref[...], (((1,), (1,)), ((), ())),
                             preferred_element_type=jnp.float32) + b3_ref[...]
    m = jnp.max(logits, axis=1, keepdims=True)
    lse = m + jnp.log(jnp.sum(jnp.exp(logits - m), axis=1, keepdims=True))
    out_ref[...] = (logits - lse)[:N, :C]


_tc1a = pl.pallas_call(_tc1a_body, out_shape=jax.ShapeDtypeStruct((NP, D), jnp.float32))
_tc1b = pl.pallas_call(_tc1b_body, out_shape=jax.ShapeDtypeStruct((NP, D), jnp.float32))
_tc2 = pl.pallas_call(_tc2_body, out_shape=jax.ShapeDtypeStruct((NP, D), jnp.float32))
_tc3 = pl.pallas_call(_tc3_body, out_shape=jax.ShapeDtypeStruct((N, C), jnp.float32))


def kernel(x, edge_index, W1, b1, W2, b2, W3, b3):
    edges = jnp.pad(edge_index.astype(jnp.int32).reshape(2, E // CHUNK, CHUNK),
                    ((0, 0), (0, NW * CPW - E // CHUNK), (0, 0)),
                    constant_values=DUMMY)
    zeros_d = jnp.zeros((NP, D), jnp.float32)
    zeros_w = jnp.zeros((NP, WDEG), jnp.float32)
    ones_w = jnp.ones((CHUNK, WDEG), jnp.float32)

    h1 = _tc1a(x, W1, b1.reshape(1, D))
    degp = _deg_sc(edges, ones_w, zeros_w)
    s1 = _tc1b(h1, degp)
    p1 = _agg_sc(s1, edges, zeros_d)
    s2 = _tc2(p1, s1, degp, W2, b2.reshape(1, D))
    p2 = _agg_sc(s2, edges, zeros_d)
    w3p = jnp.pad(W3, ((0, CP - C), (0, 0)))
    b3p = jnp.concatenate(
        [b3, jnp.full((CP - C,), -1e30, jnp.float32)]).reshape(1, CP)
    return _tc3(p2, s2, degp, w3p, b3p)


# speedup vs baseline: 1.0363x; 1.0213x over previous
import functools

import jax
import jax.numpy as jnp
from jax import lax
from jax.experimental import pallas as pl
from jax.experimental.pallas import tpu as pltpu
from jax.experimental.pallas import tpu_sc as plsc

N = 10000
NP = 10240
D_IN = 128
D = 64
C = 40
CP = 64
E = 320000
NC, NS = 2, 16
NW = NC * NS
CHUNK = 128
CPW = 80
E_PAD = NW * CPW * CHUNK
DUMMY = N
WDEG = 16
RPT = NP // NS

_MESH = plsc.VectorSubcoreMesh(core_axis_name="c", subcore_axis_name="s")


@functools.partial(
    pl.kernel,
    out_type=jax.ShapeDtypeStruct((NP, NC * WDEG), jnp.float32),
    mesh=_MESH,
    scratch_types=[
        pltpu.VMEM((CPW, CHUNK), jnp.int32),
        pltpu.VMEM((CHUNK, WDEG), jnp.float32),
        pltpu.VMEM_SHARED((NP, WDEG), jnp.float32),
        pltpu.SemaphoreType.DMA,
    ],
    compiler_params=pltpu.CompilerParams(use_tc_tiling_on_sc=False),
)
def _deg_sc(edges_hbm, ones_hbm, zeros_hbm, out_hbm, row_v, ones_v, acc, ssem):
    c = lax.axis_index("c")
    s = lax.axis_index("s")
    w = s * NC + c
    pltpu.sync_copy(edges_hbm.at[0, pl.ds(w * CPW, CPW)], row_v)
    pltpu.sync_copy(ones_hbm, ones_v)
    pltpu.sync_copy(zeros_hbm.at[pl.ds(s * RPT, RPT)], acc.at[pl.ds(s * RPT, RPT)])
    plsc.subcore_barrier()

    def body(j, carry):
        pltpu.async_copy(ones_v, acc.at[row_v.at[j]], ssem, add=True)
        return carry

    lax.fori_loop(0, CPW, body, 0)

    def drain(j, carry):
        pltpu.make_async_copy(ones_v, acc.at[row_v.at[0]], ssem).wait()
        return carry

    lax.fori_loop(0, CPW, drain, 0)
    plsc.subcore_barrier()
    pltpu.sync_copy(acc.at[pl.ds(s * RPT, RPT)],
                    out_hbm.at[pl.ds(s * RPT, RPT), pl.ds(c * WDEG, WDEG)])


@functools.partial(
    pl.kernel,
    out_type=jax.ShapeDtypeStruct((NP, NC * D), jnp.float32),
    mesh=_MESH,
    scratch_types=[
        pltpu.VMEM((CPW, CHUNK), jnp.int32),
        pltpu.VMEM((CPW, CHUNK), jnp.int32),
        pltpu.VMEM((CHUNK, D), jnp.float32),
        pltpu.VMEM((CHUNK, D), jnp.float32),
        pltpu.VMEM_SHARED((NP, D), jnp.float32),
        pltpu.VMEM_SHARED((NP, D), jnp.float32),
        pltpu.SemaphoreType.DMA,
        pltpu.SemaphoreType.DMA,
    ],
    compiler_params=pltpu.CompilerParams(use_tc_tiling_on_sc=False),
)
def _agg_sc(s_hbm, edges_hbm, zeros_hbm, out_hbm,
            row_v, col_v, bufa, bufb, acc, s_sp, sema, semb):
    c = lax.axis_index("c")
    s = lax.axis_index("s")
    w = s * NC + c
    pltpu.sync_copy(edges_hbm.at[0, pl.ds(w * CPW, CPW)], row_v)
    pltpu.sync_copy(edges_hbm.at[1, pl.ds(w * CPW, CPW)], col_v)
    pltpu.sync_copy(zeros_hbm.at[pl.ds(s * RPT, RPT)], acc.at[pl.ds(s * RPT, RPT)])
    pltpu.sync_copy(s_hbm.at[pl.ds(s * RPT, RPT)], s_sp.at[pl.ds(s * RPT, RPT)])
    plsc.subcore_barrier()

    def wait_g(buf, sem):
        pltpu.make_async_copy(s_sp.at[row_v.at[0]], buf, sem).wait()

    def prefetch(j, buf, sem):
        pltpu.async_copy(s_sp.at[row_v.at[j]], buf, sem)

    prefetch(0, bufa, sema)
    prefetch(1, bufb, semb)

    def body(i, carry):
        j = 2 * i
        wait_g(bufa, sema)
        pltpu.sync_copy(bufa, acc.at[col_v.at[j]], add=True)
        prefetch(j + 2, bufa, sema)
        wait_g(bufb, semb)
        pltpu.sync_copy(bufb, acc.at[col_v.at[j + 1]], add=True)
        prefetch(j + 3, bufb, semb)
        return carry

    lax.fori_loop(0, CPW // 2 - 1, body, 0)
    wait_g(bufa, sema)
    pltpu.sync_copy(bufa, acc.at[col_v.at[CPW - 2]], add=True)
    wait_g(bufb, semb)
    pltpu.sync_copy(bufb, acc.at[col_v.at[CPW - 1]], add=True)
    plsc.subcore_barrier()
    pltpu.sync_copy(acc.at[pl.ds(s * RPT, RPT)],
                    out_hbm.at[pl.ds(s * RPT, RPT), pl.ds(c * D, D)])


def _dis(degp_ref):
    deg = degp_ref[:, 0:1] + degp_ref[:, WDEG:WDEG + 1] + 1.0
    return lax.rsqrt(deg)


def _tc1a_body(x_ref, w1_ref, b1_ref, h_ref):
    h_ref[0:N] = lax.dot_general(
        x_ref[...], w1_ref[...], (((1,), (1,)), ((), ())),
        preferred_element_type=jnp.float32) + b1_ref[...]
    h_ref[N:] = jnp.zeros((NP - N, D), jnp.float32)


def _tc1b_body(h_ref, degp_ref, s1_ref):
    s1_ref[...] = h_ref[...] * _dis(degp_ref)


def _tc2_body(p_ref, s1_ref, degp_ref, w2_ref, b2_ref, s2_ref):
    dis = _dis(degp_ref)
    h1 = jnp.maximum(
        dis * (p_ref[:, 0:D] + p_ref[:, D:2 * D] + s1_ref[...]), 0.0)
    h = lax.dot_general(h1, w2_ref[...], (((1,), (1,)), ((), ())),
                        preferred_element_type=jnp.float32) + b2_ref[...]
    s2_ref[...] = h * dis


def _tc3_body(p_ref, s2_ref, degp_ref, w3_ref, b3_ref, out_ref):
    h2 = _dis(degp_ref)[0:N] * (
        p_ref[0:N, 0:D] + p_ref[0:N, D:2 * D] + s2_ref[0:N])
    logits = lax.dot_general(h2, w3_ref[...], (((1,), (1,)), ((), ())),
                             preferred_element_type=jnp.float32) + b3_ref[...]
    m = jnp.max(logits, axis=1, keepdims=True)
    lse = m + jnp.log(jnp.sum(jnp.exp(logits - m), axis=1, keepdims=True))
    out_ref[...] = (logits - lse)[:, :C]


_tc1a = pl.pallas_call(_tc1a_body, out_shape=jax.ShapeDtypeStruct((NP, D), jnp.float32))
_tc1b = pl.pallas_call(_tc1b_body, out_shape=jax.ShapeDtypeStruct((NP, D), jnp.float32))
_tc2 = pl.pallas_call(_tc2_body, out_shape=jax.ShapeDtypeStruct((NP, D), jnp.float32))
_tc3 = pl.pallas_call(_tc3_body, out_shape=jax.ShapeDtypeStruct((N, C), jnp.float32))


def kernel(x, edge_index, W1, b1, W2, b2, W3, b3):
    edges = jnp.pad(edge_index.astype(jnp.int32).reshape(2, E // CHUNK, CHUNK),
                    ((0, 0), (0, NW * CPW - E // CHUNK), (0, 0)),
                    constant_values=DUMMY)
    zeros_d = jnp.zeros((NP, D), jnp.float32)
    zeros_w = jnp.zeros((NP, WDEG), jnp.float32)
    ones_w = jnp.ones((CHUNK, WDEG), jnp.float32)

    h1 = _tc1a(x, W1, b1.reshape(1, D))
    degp = _deg_sc(edges, ones_w, zeros_w)
    s1 = _tc1b(h1, degp)
    p1 = _agg_sc(s1, edges, zeros_d)
    s2 = _tc2(p1, s1, degp, W2, b2.reshape(1, D))
    p2 = _agg_sc(s2, edges, zeros_d)
    w3p = jnp.pad(W3, ((0, CP - C), (0, 0)))
    b3p = jnp.concatenate(
        [b3, jnp.full((CP - C,), -1e30, jnp.float32)]).reshape(1, CP)
    return _tc3(p2, s2, degp, w3p, b3p)


# speedup vs baseline: 1.0528x; 1.0159x over previous
import functools

import jax
import jax.numpy as jnp
from jax import lax
from jax.experimental import pallas as pl
from jax.experimental.pallas import tpu as pltpu
from jax.experimental.pallas import tpu_sc as plsc

N = 10000
NP = 10240
D_IN = 128
D = 64
C = 40
CP = 64
E = 320000
NC, NS = 2, 16
NW = NC * NS
CHUNK = 128
CPW = 80
E_PAD = NW * CPW * CHUNK
DUMMY = N
WDEG = 8
RPT = NP // NS

_MESH = plsc.VectorSubcoreMesh(core_axis_name="c", subcore_axis_name="s")


@functools.partial(
    pl.kernel,
    out_type=jax.ShapeDtypeStruct((NP, NC * WDEG), jnp.float32),
    mesh=_MESH,
    scratch_types=[
        pltpu.VMEM((CPW, CHUNK), jnp.int32),
        pltpu.VMEM((CHUNK, WDEG), jnp.float32),
        pltpu.VMEM_SHARED((NP, WDEG), jnp.float32),
        pltpu.SemaphoreType.DMA,
    ],
    compiler_params=pltpu.CompilerParams(use_tc_tiling_on_sc=False),
)
def _deg_sc(edges_hbm, ones_hbm, zeros_hbm, out_hbm, row_v, ones_v, acc, ssem):
    c = lax.axis_index("c")
    s = lax.axis_index("s")
    w = s * NC + c
    pltpu.sync_copy(edges_hbm.at[0, pl.ds(w * CPW, CPW)], row_v)
    pltpu.sync_copy(ones_hbm, ones_v)
    pltpu.sync_copy(zeros_hbm.at[pl.ds(s * RPT, RPT)], acc.at[pl.ds(s * RPT, RPT)])
    plsc.subcore_barrier()

    def body(j, carry):
        pltpu.async_copy(ones_v, acc.at[row_v.at[j]], ssem, add=True)
        return carry

    lax.fori_loop(0, CPW, body, 0)

    def drain(j, carry):
        pltpu.make_async_copy(ones_v, acc.at[row_v.at[0]], ssem).wait()
        return carry

    lax.fori_loop(0, CPW, drain, 0)
    plsc.subcore_barrier()
    pltpu.sync_copy(acc.at[pl.ds(s * RPT, RPT)],
                    out_hbm.at[pl.ds(s * RPT, RPT), pl.ds(c * WDEG, WDEG)])


@functools.partial(
    pl.kernel,
    out_type=jax.ShapeDtypeStruct((NP, NC * D), jnp.float32),
    mesh=_MESH,
    scratch_types=[
        pltpu.VMEM((CPW, CHUNK), jnp.int32),
        pltpu.VMEM((CPW, CHUNK), jnp.int32),
        pltpu.VMEM((CHUNK, D), jnp.float32),
        pltpu.VMEM((CHUNK, D), jnp.float32),
        pltpu.VMEM_SHARED((NP, D), jnp.float32),
        pltpu.VMEM_SHARED((NP, D), jnp.float32),
        pltpu.SemaphoreType.DMA,
        pltpu.SemaphoreType.DMA,
    ],
    compiler_params=pltpu.CompilerParams(use_tc_tiling_on_sc=False),
)
def _agg_sc(s_hbm, edges_hbm, zeros_hbm, out_hbm,
            row_v, col_v, bufa, bufb, acc, s_sp, sema, semb):
    c = lax.axis_index("c")
    s = lax.axis_index("s")
    w = s * NC + c
    pltpu.sync_copy(edges_hbm.at[0, pl.ds(w * CPW, CPW)], row_v)
    pltpu.sync_copy(edges_hbm.at[1, pl.ds(w * CPW, CPW)], col_v)
    pltpu.sync_copy(zeros_hbm.at[pl.ds(s * RPT, RPT)], acc.at[pl.ds(s * RPT, RPT)])
    pltpu.sync_copy(s_hbm.at[pl.ds(s * RPT, RPT)], s_sp.at[pl.ds(s * RPT, RPT)])
    plsc.subcore_barrier()

    def wait_g(buf, sem):
        pltpu.make_async_copy(s_sp.at[row_v.at[0]], buf, sem).wait()

    def prefetch(j, buf, sem):
        pltpu.async_copy(s_sp.at[row_v.at[j]], buf, sem)

    prefetch(0, bufa, sema)
    prefetch(1, bufb, semb)

    def body(i, carry):
        j = 2 * i
        wait_g(bufa, sema)
        pltpu.sync_copy(bufa, acc.at[col_v.at[j]], add=True)
        prefetch(j + 2, bufa, sema)
        wait_g(bufb, semb)
        pltpu.sync_copy(bufb, acc.at[col_v.at[j + 1]], add=True)
        prefetch(j + 3, bufb, semb)
        return carry

    lax.fori_loop(0, CPW // 2 - 1, body, 0)
    wait_g(bufa, sema)
    pltpu.sync_copy(bufa, acc.at[col_v.at[CPW - 2]], add=True)
    wait_g(bufb, semb)
    pltpu.sync_copy(bufb, acc.at[col_v.at[CPW - 1]], add=True)
    plsc.subcore_barrier()
    pltpu.sync_copy(acc.at[pl.ds(s * RPT, RPT)],
                    out_hbm.at[pl.ds(s * RPT, RPT), pl.ds(c * D, D)])


def _dis(degp_ref):
    deg = degp_ref[:, 0:1] + degp_ref[:, WDEG:WDEG + 1] + 1.0
    return lax.rsqrt(deg)


def _tc1a_body(x_ref, w1_ref, b1_ref, h_ref):
    h_ref[0:N] = lax.dot_general(
        x_ref[...], w1_ref[...], (((1,), (1,)), ((), ())),
        preferred_element_type=jnp.float32) + b1_ref[...]
    h_ref[N:] = jnp.zeros((NP - N, D), jnp.float32)


def _tc1b_body(h_ref, degp_ref, s1_ref):
    s1_ref[...] = h_ref[...] * _dis(degp_ref)


def _tc2_body(p_ref, s1_ref, degp_ref, w2_ref, b2_ref, s2_ref):
    dis = _dis(degp_ref)
    h1 = jnp.maximum(
        dis * (p_ref[:, 0:D] + p_ref[:, D:2 * D] + s1_ref[...]), 0.0)
    h = lax.dot_general(h1, w2_ref[...], (((1,), (1,)), ((), ())),
                        preferred_element_type=jnp.float32) + b2_ref[...]
    s2_ref[...] = h * dis


def _tc3_body(p_ref, s2_ref, degp_ref, w3_ref, b3_ref, out_ref):
    h2 = _dis(degp_ref)[0:N] * (
        p_ref[0:N, 0:D] + p_ref[0:N, D:2 * D] + s2_ref[0:N])
    logits = lax.dot_general(h2, w3_ref[...], (((1,), (1,)), ((), ())),
                             preferred_element_type=jnp.float32) + b3_ref[...]
    m = jnp.max(logits, axis=1, keepdims=True)
    lse = m + jnp.log(jnp.sum(jnp.exp(logits - m), axis=1, keepdims=True))
    out_ref[...] = (logits - lse)[:, :C]


_tc1a = pl.pallas_call(_tc1a_body, out_shape=jax.ShapeDtypeStruct((NP, D), jnp.float32))
_tc1b = pl.pallas_call(_tc1b_body, out_shape=jax.ShapeDtypeStruct((NP, D), jnp.float32))
_tc2 = pl.pallas_call(_tc2_body, out_shape=jax.ShapeDtypeStruct((NP, D), jnp.float32))
_tc3 = pl.pallas_call(_tc3_body, out_shape=jax.ShapeDtypeStruct((N, C), jnp.float32))


def kernel(x, edge_index, W1, b1, W2, b2, W3, b3):
    edges = jnp.pad(edge_index.astype(jnp.int32).reshape(2, E // CHUNK, CHUNK),
                    ((0, 0), (0, NW * CPW - E // CHUNK), (0, 0)),
                    constant_values=DUMMY)
    zeros_d = jnp.zeros((NP, D), jnp.float32)
    zeros_w = jnp.zeros((NP, WDEG), jnp.float32)
    ones_w = jnp.ones((CHUNK, WDEG), jnp.float32)

    h1 = _tc1a(x, W1, b1.reshape(1, D))
    degp = _deg_sc(edges, ones_w, zeros_w)
    s1 = _tc1b(h1, degp)
    p1 = _agg_sc(s1, edges, zeros_d)
    s2 = _tc2(p1, s1, degp, W2, b2.reshape(1, D))
    p2 = _agg_sc(s2, edges, zeros_d)
    w3p = jnp.pad(W3, ((0, CP - C), (0, 0)))
    b3p = jnp.concatenate(
        [b3, jnp.full((CP - C,), -1e30, jnp.float32)]).reshape(1, CP)
    return _tc3(p2, s2, degp, w3p, b3p)
